# baseline (speedup 1.0000x reference)
import functools
import jax
import jax.numpy as jnp
import numpy as np
from jax import lax
from jax.experimental import pallas as pl
from jax.experimental.pallas import tpu as pltpu
from jax.experimental.pallas import tpu_sc as plsc

T_STEPS = 32
CH = 1024
NC = 2
NS = 16
NW = NC * NS
LANES = 16


def _enc_body(x_hbm, out_hbm, xv, tv, ot):
    B = x_hbm.shape[0]
    F = x_hbm.shape[1]
    n_per_b = F // CH
    total = B * n_per_b
    k_max = (total + NW - 1) // NW

    wid = lax.axis_index("s") * NC + lax.axis_index("c")

    ones = jnp.full((LANES,), 1.0, jnp.float32)
    zeros = jnp.zeros((LANES,), jnp.float32)
    lane_iota = lax.broadcasted_iota(jnp.int32, (LANES,), 0)

    def _zero_row(t, _):
        def _zero_vec(j, _):
            ot[t, pl.ds(j * LANES, LANES)] = zeros
            return None
        return lax.fori_loop(0, CH // LANES, _zero_vec, None)

    lax.fori_loop(0, T_STEPS, _zero_row, None)

    def _chunk(k, _):
        g = k * NW + wid

        @pl.when(g < total)
        def _():
            b = g // n_per_b
            f0 = (g % n_per_b) * CH
            pltpu.sync_copy(x_hbm.at[b, pl.ds(f0, CH)], xv)

            def _scat(i, _):
                xr = xv[pl.ds(i * LANES, LANES)]
                xc = jnp.clip(xr, 0.0, 1.0)
                t = (T_STEPS * (1.0 - xc)).astype(jnp.int32)
                t = jnp.clip(t, 0, T_STEPS - 1)
                tv[pl.ds(i * LANES, LANES)] = t
                fvec = i * LANES + lane_iota
                plsc.store_scatter(ot, [t, fvec], ones)
                return None

            lax.fori_loop(0, CH // LANES, _scat, None)

            pltpu.sync_copy(ot, out_hbm.at[b, :, pl.ds(f0, CH)])

            def _clear(i, _):
                t = tv[pl.ds(i * LANES, LANES)]
                fvec = i * LANES + lane_iota
                plsc.store_scatter(ot, [t, fvec], zeros)
                return None

            lax.fori_loop(0, CH // LANES, _clear, None)

        return None

    lax.fori_loop(0, k_max, _chunk, None)


def kernel(x):
    B = x.shape[0]
    rest = x.shape[1:]
    F = int(np.prod(rest))
    assert F % CH == 0
    x2 = x.reshape(B, F)
    enc = functools.partial(
        pl.kernel,
        out_type=jax.ShapeDtypeStruct((B, T_STEPS, F), jnp.float32),
        mesh=plsc.VectorSubcoreMesh(core_axis_name="c", subcore_axis_name="s"),
        scratch_types=[
            pltpu.VMEM((CH,), jnp.float32),
            pltpu.VMEM((CH,), jnp.int32),
            pltpu.VMEM((T_STEPS, CH), jnp.float32),
        ],
        compiler_params=pltpu.CompilerParams(use_tc_tiling_on_sc=False, needs_layout_passes=False),
    )(_enc_body)
    out = enc(x2)
    return out.reshape((B, T_STEPS) + tuple(rest))


# speedup vs baseline: 1.0369x; 1.0369x over previous
import functools
import jax
import jax.numpy as jnp
import numpy as np
from jax import lax
from jax.experimental import pallas as pl
from jax.experimental.pallas import tpu as pltpu
from jax.experimental.pallas import tpu_sc as plsc

T_STEPS = 32
CH = 1024
NC = 2
NS = 16
NW = NC * NS
LANES = 16


def _enc_body(x_hbm, out_hbm, xv, tv, ot, sems):
    B = x_hbm.shape[0]
    F = x_hbm.shape[1]
    n_per_b = F // CH
    total = B * n_per_b

    wid = lax.axis_index("s") * NC + lax.axis_index("c")
    nk = (total - wid + NW - 1) // NW

    ones = jnp.full((LANES,), 1.0, jnp.float32)
    zeros = jnp.zeros((LANES,), jnp.float32)
    lane_iota = lax.broadcasted_iota(jnp.int32, (LANES,), 0)

    def _zero_vec(i, _):
        t = i // (CH // LANES)
        j = i % (CH // LANES)
        ot[0, t, pl.ds(j * LANES, LANES)] = zeros
        ot[1, t, pl.ds(j * LANES, LANES)] = zeros
        return None

    lax.fori_loop(0, T_STEPS * (CH // LANES), _zero_vec, None)

    def _out_slices(g, half):
        b = g // n_per_b
        f0 = (g % n_per_b) * CH
        return ot.at[half], out_hbm.at[b, :, pl.ds(f0, CH)]

    def _process(k, half):
        g = k * NW + wid

        @pl.when(g < total)
        def _():
            src, dst = _out_slices(g, half)

            @pl.when(k >= 2)
            def _():
                pltpu.make_async_copy(src, dst, sems.at[half]).wait()
                for i in range(CH // LANES):
                    t = tv[half, pl.ds(i * LANES, LANES)]
                    plsc.store_scatter(src, [t, i * LANES + lane_iota], zeros)

            b = g // n_per_b
            f0 = (g % n_per_b) * CH
            pltpu.sync_copy(x_hbm.at[b, pl.ds(f0, CH)], xv.at[half])

            for i in range(CH // LANES):
                xr = xv[half, pl.ds(i * LANES, LANES)]
                xc = jnp.clip(xr, 0.0, 1.0)
                t = (T_STEPS * (1.0 - xc)).astype(jnp.int32)
                t = jnp.clip(t, 0, T_STEPS - 1)
                tv[half, pl.ds(i * LANES, LANES)] = t
                plsc.store_scatter(src, [t, i * LANES + lane_iota], ones)

            pltpu.async_copy(src, dst, sems.at[half])

        return None

    def _pair(kk, _):
        _process(kk * 2, 0)
        _process(kk * 2 + 1, 1)
        return None

    n_pairs = (nk + 1) // 2
    lax.fori_loop(0, n_pairs, _pair, None)

    @pl.when(nk >= 1)
    def _():
        g = (nk - 1) * NW + wid
        src, dst = _out_slices(g, (nk - 1) % 2)
        pltpu.make_async_copy(src, dst, sems.at[(nk - 1) % 2]).wait()

    @pl.when(nk >= 2)
    def _():
        g = (nk - 2) * NW + wid
        src, dst = _out_slices(g, (nk - 2) % 2)
        pltpu.make_async_copy(src, dst, sems.at[(nk - 2) % 2]).wait()


def kernel(x):
    B = x.shape[0]
    rest = x.shape[1:]
    F = int(np.prod(rest))
    assert F % CH == 0
    x2 = x.reshape(B, F)
    enc = functools.partial(
        pl.kernel,
        out_type=jax.ShapeDtypeStruct((B, T_STEPS, F), jnp.float32),
        mesh=plsc.VectorSubcoreMesh(core_axis_name="c", subcore_axis_name="s"),
        scratch_types=[
            pltpu.VMEM((2, CH), jnp.float32),
            pltpu.VMEM((2, CH), jnp.int32),
            pltpu.VMEM((2, T_STEPS, CH), jnp.float32),
            pltpu.SemaphoreType.DMA((2,)),
        ],
        compiler_params=pltpu.CompilerParams(
            use_tc_tiling_on_sc=False, needs_layout_passes=False
        ),
    )(_enc_body)
    out = enc(x2)
    return out.reshape((B, T_STEPS) + tuple(rest))


# speedup vs baseline: 4.7077x; 4.5400x over previous
import jax
import jax.numpy as jnp
import numpy as np
from jax.experimental import pallas as pl

T_STEPS = 32
T_CHUNK = 4


def _body(x_ref, o_ref):
    xb = x_ref[...]
    xc = jnp.clip(xb, 0.0, 1.0)
    t = (T_STEPS * (1.0 - xc)).astype(jnp.int32)
    t = jnp.clip(t, 0, T_STEPS - 1)
    C, H, W = xb.shape[1:]
    t_base = pl.program_id(1) * T_CHUNK
    tio = t_base + jax.lax.broadcasted_iota(
        jnp.int32, (1, T_CHUNK, C, H, W), 1
    )
    o_ref[...] = (tio == t[:, None]).astype(jnp.float32)


def kernel(x):
    B, C, H, W = x.shape
    out = pl.pallas_call(
        _body,
        grid=(B, T_STEPS // T_CHUNK),
        in_specs=[pl.BlockSpec((1, C, H, W), lambda b, tc: (b, 0, 0, 0))],
        out_specs=pl.BlockSpec(
            (1, T_CHUNK, C, H, W), lambda b, tc: (b, tc, 0, 0, 0)
        ),
        out_shape=jax.ShapeDtypeStruct((B, T_STEPS, C, H, W), jnp.float32),
    )(x)
    return out


# speedup vs baseline: 5.7941x; 1.2308x over previous
import jax
import jax.numpy as jnp
import numpy as np
from jax.experimental import pallas as pl

T_STEPS = 32
T_CHUNK = 8


def _body(x_ref, o_ref):
    xb = x_ref[...]
    xc = jnp.clip(xb, 0.0, 1.0)
    t = (T_STEPS * (1.0 - xc)).astype(jnp.int32)
    t = jnp.clip(t, 0, T_STEPS - 1)
    C, H, W = xb.shape[1:]
    t_base = pl.program_id(1) * T_CHUNK
    tio = t_base + jax.lax.broadcasted_iota(
        jnp.int32, (1, T_CHUNK, C, H, W), 1
    )
    o_ref[...] = (tio == t[:, None]).astype(jnp.float32)


def kernel(x):
    B, C, H, W = x.shape
    out = pl.pallas_call(
        _body,
        grid=(B, T_STEPS // T_CHUNK),
        in_specs=[pl.BlockSpec((1, C, H, W), lambda b, tc: (b, 0, 0, 0))],
        out_specs=pl.BlockSpec(
            (1, T_CHUNK, C, H, W), lambda b, tc: (b, tc, 0, 0, 0)
        ),
        out_shape=jax.ShapeDtypeStruct((B, T_STEPS, C, H, W), jnp.float32),
    )(x)
    return out


# speedup vs baseline: 6.0808x; 1.0495x over previous
import jax
import jax.numpy as jnp
import numpy as np
from jax.experimental import pallas as pl

T_STEPS = 32
T_CHUNK = 16


def _body(x_ref, o_ref):
    xb = x_ref[...]
    xc = jnp.clip(xb, 0.0, 1.0)
    t = (T_STEPS * (1.0 - xc)).astype(jnp.int32)
    t = jnp.clip(t, 0, T_STEPS - 1)
    C, H, W = xb.shape[1:]
    t_base = pl.program_id(1) * T_CHUNK
    tio = t_base + jax.lax.broadcasted_iota(
        jnp.int32, (1, T_CHUNK, C, H, W), 1
    )
    o_ref[...] = (tio == t[:, None]).astype(jnp.float32)


def kernel(x):
    B, C, H, W = x.shape
    out = pl.pallas_call(
        _body,
        grid=(B, T_STEPS // T_CHUNK),
        in_specs=[pl.BlockSpec((1, C, H, W), lambda b, tc: (b, 0, 0, 0))],
        out_specs=pl.BlockSpec(
            (1, T_CHUNK, C, H, W), lambda b, tc: (b, tc, 0, 0, 0)
        ),
        out_shape=jax.ShapeDtypeStruct((B, T_STEPS, C, H, W), jnp.float32),
    )(x)
    return out
